# baseline (speedup 1.0000x reference)
import functools

import jax
import jax.numpy as jnp
from jax import lax
from jax.experimental import pallas as pl
from jax.experimental.pallas import tpu as pltpu
from jax.experimental.pallas import tpu_sc as plsc

N, D, G = 32768, 320, 16384
DP = 384
RB = 4096
SB = 256
NSB = RB // SB
NB = N // RB

NC, NS = 2, 16
NW = NC * NS
GP_W = G // NW
CH = 64
NCH = GP_W // CH
LANES = 16


def _sufsum_body(x_ref, ss_ref, carry_ref):
    b = pl.program_id(0)

    @pl.when(b == 0)
    def _():
        carry_ref[...] = jnp.zeros_like(carry_ref)
        ss_ref[...] = jnp.zeros((RB, DP), jnp.float32)

    @pl.when(b > 0)
    def _():
        carry = carry_ref[...]
        for s in reversed(range(NSB)):
            xs = x_ref[pl.ds(s * SB, SB), :]
            suf = xs
            sh = 1
            while sh < SB:
                suf = suf + jnp.concatenate(
                    [suf[sh:], jnp.zeros((sh, D), jnp.float32)], axis=0)
                sh *= 2
            ss_ref[pl.ds(s * SB, SB), : D] = suf + carry
            carry = carry + suf[0:1]
        carry_ref[...] = carry


def _sufsum_tc(x):
    return pl.pallas_call(
        _sufsum_body,
        grid=(NB + 1,),
        in_specs=[pl.BlockSpec(
            (RB, D), lambda b: (jnp.minimum(NB - b, NB - 1), 0))],
        out_specs=pl.BlockSpec((RB, DP), lambda b: (NB - b, 0)),
        out_shape=jax.ShapeDtypeStruct((N + 8, DP), jnp.float32),
        scratch_shapes=[pltpu.VMEM((1, D), jnp.float32)],
    )(x)


def _gather_sub_sc(cs, prep_idx):
    mesh = plsc.VectorSubcoreMesh(core_axis_name="c", subcore_axis_name="s")

    @functools.partial(
        pl.kernel,
        out_type=jax.ShapeDtypeStruct((G, D), jnp.float32),
        mesh=mesh,
        compiler_params=pltpu.CompilerParams(use_tc_tiling_on_sc=True),
        scratch_types=[
            pltpu.VMEM((NCH, CH), jnp.int32),
            pltpu.VMEM((NCH, CH), jnp.int32),
            pltpu.VMEM((CH, DP), jnp.float32),
            pltpu.VMEM((CH, DP), jnp.float32),
            pltpu.VMEM((CH, DP), jnp.float32),
            pltpu.VMEM((CH, DP), jnp.float32),
            pltpu.VMEM((CH, D), jnp.float32),
            pltpu.SemaphoreType.DMA,
            pltpu.SemaphoreType.DMA,
            pltpu.SemaphoreType.DMA,
            pltpu.SemaphoreType.DMA,
        ],
    )
    def k(cs_hbm, pidx_hbm, out_hbm, idx_s, idx_e, buf_s0, buf_e0, buf_s1,
          buf_e1, buf_d, sem_s0, sem_e0, sem_s1, sem_e1):
        wid = lax.axis_index("s") * NC + lax.axis_index("c")
        gbase = wid * GP_W
        icps = []
        for c in range(NCH):
            icps.append(pltpu.async_copy(
                pidx_hbm.at[pl.ds(gbase + c * CH, CH)], idx_s.at[c], sem_s0))
            icps.append(pltpu.async_copy(
                pidx_hbm.at[pl.ds(G + gbase + c * CH, CH)], idx_e.at[c],
                sem_s0))
        for cp in icps:
            cp.wait()

        sbufs = (buf_s0, buf_s1)
        ebufs = (buf_e0, buf_e1)
        ssems = (sem_s0, sem_s1)
        esems = (sem_e0, sem_e1)
        cps = [None, None]

        def start(c):
            i = c % 2
            return (
                pltpu.async_copy(cs_hbm.at[idx_s.at[c]], sbufs[i], ssems[i]),
                pltpu.async_copy(cs_hbm.at[idx_e.at[c]], ebufs[i], esems[i]),
            )

        cps[0] = start(0)
        for c in range(NCH):
            i = c % 2
            if c + 1 < NCH:
                cps[(c + 1) % 2] = start(c + 1)
            cps[i][0].wait()
            cps[i][1].wait()
            bs, be = sbufs[i], ebufs[i]

            @plsc.parallel_loop(0, CH, 1, unroll=4)
            def _(r):
                for j in range(D // LANES):
                    sl = pl.ds(j * LANES, LANES)
                    buf_d[r, sl] = bs[r, sl] - be[r, sl]
            pltpu.sync_copy(buf_d, out_hbm.at[pl.ds(gbase + c * CH, CH)])

    return k(cs, prep_idx)


def kernel(InputVector, wordGroupsID):
    ss = _sufsum_tc(InputVector)
    prep_idx = jnp.concatenate([wordGroupsID[:, 0], wordGroupsID[:, 1] + 1])
    return _gather_sub_sc(ss, prep_idx)
